# baseline (speedup 1.0000x reference)
import functools

import jax
import jax.numpy as jnp
from jax import lax
from jax.experimental import pallas as pl
from jax.experimental.pallas import tpu as pltpu
from jax.experimental.pallas import tpu_sc as plsc

_N = 1_000_000
_BATCH = 16384
_D = 64
_BETA = 0.9
_NC = 2
_NS = 16
_NW = _NC * _NS
_BPW = _BATCH // _NW
_L = 16
_NG = _BPW // _L
_CH = 128
_NT = _BPW * _D // _CH
_LO = 0.0001
_HI = 1.0 - 0.0001
_WAVE = 8


@functools.partial(
    pl.kernel,
    mesh=plsc.VectorSubcoreMesh(core_axis_name="c", subcore_axis_name="s"),
    scratch_types=[
        pltpu.VMEM((_BPW,), jnp.int32),
        pltpu.VMEM((_D, _BPW), jnp.float32),
        pltpu.VMEM((_NT, _CH), jnp.int32),
        pltpu.VMEM((_BPW * _D,), jnp.float32),
        pltpu.SemaphoreType.DMA,
    ],
    compiler_params=pltpu.CompilerParams(use_tc_tiling_on_sc=False),
)
def _sc_ema_flat(
    probs_t_hbm, idx_hbm, latf_ref,
    idx_v, probs_tv, addr_v, vals_v, sem,
):
    wid = lax.axis_index("s") * _NC + lax.axis_index("c")
    base = wid * _BPW

    pltpu.sync_copy(idx_hbm.at[pl.ds(base, _BPW)], idx_v)
    pltpu.sync_copy(probs_t_hbm.at[:, pl.ds(base, _BPW)], probs_tv)

    def addr_body(g, carry):
        iv = idx_v[pl.ds(g * _L, _L)]
        for h in range(8):
            for k in range(8):
                c = 8 * h + k
                addr_v[g * 8 + h, pl.ds(k * _L, _L)] = iv + (c * _N)
        return carry

    lax.fori_loop(0, _NG, addr_body, 0)

    def gather_wave(w, carry):
        cps = [
            pltpu.async_copy(
                latf_ref.at[addr_v.at[w * _WAVE + t]],
                vals_v.at[pl.ds((w * _WAVE + t) * _CH, _CH)],
                sem,
            )
            for t in range(_WAVE)
        ]
        for cp in cps:
            cp.wait()
        return carry

    lax.fori_loop(0, _NT // _WAVE, gather_wave, 0)

    def group_body(g, carry):
        clipped = []
        acc = None
        for c in range(_D):
            v = probs_tv[c, pl.ds(g * _L, _L)]
            v = jnp.minimum(jnp.maximum(v, _LO), _HI)
            clipped.append(v)
            acc = v if acc is None else acc + v
        sv = (1.0 - _BETA) / acc
        for c in range(_D):
            off = (g * 8 + (c // 8)) * _CH + (c % 8) * _L
            gv = vals_v[pl.ds(off, _L)]
            vals_v[pl.ds(off, _L)] = _BETA * gv + clipped[c] * sv
        return carry

    lax.fori_loop(0, _NG, group_body, 0)

    def scatter_wave(w, carry):
        cps = [
            pltpu.async_copy(
                vals_v.at[pl.ds((w * _WAVE + t) * _CH, _CH)],
                latf_ref.at[addr_v.at[w * _WAVE + t]],
                sem,
            )
            for t in range(_WAVE)
        ]
        for cp in cps:
            cp.wait()
        return carry

    lax.fori_loop(0, _NT // _WAVE, scatter_wave, 0)


def kernel(probs, index, latent):
    probs_t = probs.T
    lat_flat = latent.T.reshape(-1)
    out_ref = jax.new_ref(lat_flat)
    _sc_ema_flat(probs_t, index, out_ref)
    return out_ref[...].reshape(_D, _N).T


# speedup vs baseline: 16.0149x; 16.0149x over previous
import functools

import jax
import jax.numpy as jnp
from jax import lax
from jax.experimental import pallas as pl
from jax.experimental.pallas import tpu as pltpu
from jax.experimental.pallas import tpu_sc as plsc

_N = 1_000_000
_BATCH = 16384
_D = 64
_DP = 128
_BETA = 0.9
_NC = 2
_NS = 16
_NW = _NC * _NS
_BPW = _BATCH // _NW
_L = 16
_NSL = _D // _L
_CH = 128
_NCH = _BPW // _CH
_GPC = _CH // _L
_HB = _BPW // 2
_LO = 0.0001
_HI = 1.0 - 0.0001


@functools.partial(
    pl.kernel,
    mesh=plsc.VectorSubcoreMesh(core_axis_name="c", subcore_axis_name="s"),
    scratch_types=[
        pltpu.VMEM((_NCH, _CH), jnp.int32),
        pltpu.VMEM((_BPW, _DP), jnp.float32),
        pltpu.VMEM((_D, _HB), jnp.float32),
        pltpu.VMEM((_BPW,), jnp.float32),
        pltpu.VMEM((2, _CH, _DP), jnp.float32),
        pltpu.SemaphoreType.DMA,
        pltpu.SemaphoreType.DMA,
    ],
)
def _sc_ema_pad(
    probs_hbm, probs_t_hbm, idx_hbm, latp_ref,
    idx_v, probs_v, ptv, svv, rows_v, sem0, sem1,
):
    wid = lax.axis_index("s") * _NC + lax.axis_index("c")
    base = wid * _BPW

    pltpu.sync_copy(idx_hbm.at[wid], idx_v)
    g0 = pltpu.async_copy(latp_ref.at[idx_v.at[0]], rows_v.at[0], sem0)
    g1 = pltpu.async_copy(latp_ref.at[idx_v.at[1]], rows_v.at[1], sem1)
    pltpu.sync_copy(probs_hbm.at[pl.ds(base, _BPW)], probs_v)

    for h in range(2):
        pltpu.sync_copy(
            probs_t_hbm.at[:, pl.ds(base + h * _HB, _HB)], ptv
        )
        for lg in range(_HB // _L):
            acc = None
            for c in range(_D):
                v = ptv[c, pl.ds(lg * _L, _L)]
                v = jnp.minimum(jnp.maximum(v, _LO), _HI)
                acc = v if acc is None else acc + v
            svv[pl.ds(h * _HB + lg * _L, _L)] = (1.0 - _BETA) / acc

    def compute_chunk(q, slot):
        for lg in range(_GPC):
            g = q * _GPC + lg
            svec = svv[pl.ds(g * _L, _L)]
            for k in range(_L):
                r = g * _L + k
                s = svec[k]
                for j in range(_NSL):
                    p = probs_v[r, pl.ds(j * _L, _L)]
                    p = jnp.minimum(jnp.maximum(p, _LO), _HI)
                    gr = rows_v[slot, lg * _L + k, pl.ds(j * _L, _L)]
                    rows_v[slot, lg * _L + k, pl.ds(j * _L, _L)] = (
                        _BETA * gr + p * s
                    )

    g0.wait()
    compute_chunk(0, 0)
    s0 = pltpu.async_copy(rows_v.at[0], latp_ref.at[idx_v.at[0]], sem0)
    g1.wait()
    compute_chunk(1, 1)
    s1 = pltpu.async_copy(rows_v.at[1], latp_ref.at[idx_v.at[1]], sem1)
    s0.wait()
    pltpu.async_copy(latp_ref.at[idx_v.at[2]], rows_v.at[0], sem0).wait()
    compute_chunk(2, 0)
    s2 = pltpu.async_copy(rows_v.at[0], latp_ref.at[idx_v.at[2]], sem0)
    s1.wait()
    pltpu.async_copy(latp_ref.at[idx_v.at[3]], rows_v.at[1], sem1).wait()
    compute_chunk(3, 1)
    s3 = pltpu.async_copy(rows_v.at[1], latp_ref.at[idx_v.at[3]], sem1)
    s2.wait()
    s3.wait()


def kernel(probs, index, latent):
    idx3 = index.reshape(_NW, _NCH, _CH)
    probs_p = jnp.pad(probs, ((0, 0), (0, _DP - _D)))
    latp = jnp.pad(latent, ((0, 0), (0, _DP - _D)))
    out_ref = jax.new_ref(latp)
    _sc_ema_pad(probs_p, probs.T, idx3, out_ref)
    return out_ref[...][:, :_D]
